# baseline (speedup 1.0000x reference)
import functools

import jax
import jax.numpy as jnp
from jax import lax
from jax.experimental import pallas as pl
from jax.experimental.pallas import tpu as pltpu
from jax.experimental.pallas import tpu_sc as plsc

N = 10000
E = 320000
IN_CH = 128
HID = 32

NPAD = 10240
EPAD = 327680

NC = 2
NS = 16
NTILES = NC * NS
EPT = EPAD // NTILES
CHUNK = 128
NCHUNK = EPT // CHUNK
NSLICE = NPAD // NS
DEGW = 8

@functools.cache
def _sc_kernels():
    mesh = plsc.VectorSubcoreMesh(core_axis_name="c", subcore_axis_name="s")
    params = pltpu.CompilerParams(use_tc_tiling_on_sc=False)
    params_nl = pltpu.CompilerParams(use_tc_tiling_on_sc=False,
                                     needs_layout_passes=False)
    deg = functools.partial(
        pl.kernel,
        out_type=jax.ShapeDtypeStruct((NC, NPAD, DEGW), jnp.float32),
        mesh=mesh,
        compiler_params=params,
        scratch_types=[
            pltpu.VMEM((NCHUNK, CHUNK), jnp.int32),
            pltpu.VMEM((CHUNK, DEGW), jnp.float32),
            pltpu.VMEM_SHARED((NPAD, DEGW), jnp.float32),
            pltpu.SemaphoreType.DMA,
        ],
    )(_deg_body)
    agg = functools.partial(
        pl.kernel,
        out_type=jax.ShapeDtypeStruct((NC, NPAD, HID), jnp.float32),
        mesh=mesh,
        compiler_params=params,
        scratch_types=[
            pltpu.VMEM((NCHUNK, CHUNK), jnp.int32),
            pltpu.VMEM((NCHUNK, CHUNK), jnp.int32),
            pltpu.VMEM((CHUNK, HID), jnp.float32),
            pltpu.VMEM((CHUNK, HID), jnp.float32),
            pltpu.VMEM_SHARED((NPAD, HID), jnp.float32),
            pltpu.SemaphoreType.DMA,
            pltpu.SemaphoreType.DMA,
            pltpu.SemaphoreType.DMA,
            pltpu.SemaphoreType.DMA,
        ],
    )(_agg_body)
    edge = functools.partial(
        pl.kernel,
        out_type=[
            jax.ShapeDtypeStruct((EPAD,), jnp.float32),
            jax.ShapeDtypeStruct((EPAD,), jnp.float32),
        ],
        mesh=mesh,
        compiler_params=params_nl,
        scratch_types=[
            pltpu.VMEM((NCHUNK, CHUNK), jnp.int32),
            pltpu.VMEM((NCHUNK, CHUNK), jnp.int32),
            pltpu.VMEM((2 * HID,), jnp.float32),
            pltpu.VMEM((16,), jnp.float32),
            pltpu.VMEM((16,), jnp.float32),
            pltpu.VMEM((CHUNK, 2 * HID), jnp.float32),
            pltpu.VMEM((CHUNK, 2 * HID), jnp.float32),
            pltpu.VMEM((CHUNK, 2 * HID), jnp.float32),
            pltpu.VMEM((CHUNK, 2 * HID), jnp.float32),
            pltpu.VMEM((CHUNK, 2 * HID), jnp.float32),
            pltpu.VMEM((CHUNK,), jnp.float32),
            pltpu.VMEM((CHUNK,), jnp.float32),
            pltpu.SemaphoreType.DMA,
            pltpu.SemaphoreType.DMA,
            pltpu.SemaphoreType.DMA,
            pltpu.SemaphoreType.DMA,
        ],
    )(_edge_body)
    return deg, agg, edge


def _deg_body(dst_hbm, ones_hbm, zeros_hbm, out_hbm, dst_v, ones_v, acc_sh,
              sem):
    cid = lax.axis_index("c")
    tid = lax.axis_index("s")
    wid = cid * NS + tid
    base = tid * NSLICE
    pltpu.sync_copy(zeros_hbm.at[pl.ds(base, NSLICE)],
                    acc_sh.at[pl.ds(base, NSLICE)])
    pltpu.sync_copy(ones_hbm, ones_v)
    pltpu.sync_copy(dst_hbm.at[wid], dst_v)
    plsc.subcore_barrier()

    BATCH = 16

    def _batch(b, carry):
        for i in range(BATCH):
            pltpu.make_async_copy(
                ones_v, acc_sh.at[dst_v.at[b * BATCH + i]], sem
            ).start(add=True)
        for i in range(BATCH):
            pltpu.make_async_copy(
                ones_v, acc_sh.at[dst_v.at[b * BATCH + i]], sem
            ).wait()
        return carry

    lax.fori_loop(0, NCHUNK // BATCH, _batch, 0)

    plsc.subcore_barrier()
    pltpu.sync_copy(acc_sh.at[pl.ds(base, NSLICE)],
                    out_hbm.at[cid, pl.ds(base, NSLICE)])


def _agg_body(g_hbm, src_hbm, dst_hbm, zeros_hbm, out_hbm,
              src_v, dst_v, r0, r1, acc_sh, sg0, sg1, ss0, ss1):
    cid = lax.axis_index("c")
    tid = lax.axis_index("s")
    wid = cid * NS + tid
    base = tid * NSLICE
    pltpu.sync_copy(zeros_hbm.at[pl.ds(base, NSLICE)],
                    acc_sh.at[pl.ds(base, NSLICE)])
    pltpu.sync_copy(src_hbm.at[wid], src_v)
    pltpu.sync_copy(dst_hbm.at[wid], dst_v)
    plsc.subcore_barrier()

    def _gather(j, buf, sem):
        return pltpu.make_async_copy(g_hbm.at[src_v.at[j]], buf, sem)

    def _scatter(j, buf, sem):
        return pltpu.make_async_copy(buf, acc_sh.at[dst_v.at[j]], sem)

    _gather(0, r0, sg0).start()
    _gather(1, r1, sg1).start()

    def _pair(k, carry):
        j = 2 * k
        _gather(j, r0, sg0).wait()
        _scatter(j, r0, ss0).start(add=True)
        _gather(j + 1, r1, sg1).wait()
        _scatter(j + 1, r1, ss1).start(add=True)
        _scatter(j, r0, ss0).wait()
        _gather(j + 2, r0, sg0).start()
        _scatter(j + 1, r1, ss1).wait()
        _gather(j + 3, r1, sg1).start()
        return carry

    lax.fori_loop(0, NCHUNK // 2 - 1, _pair, 0)
    j = NCHUNK - 2
    _gather(j, r0, sg0).wait()
    pltpu.sync_copy(r0, acc_sh.at[dst_v.at[j]], add=True)
    _gather(j + 1, r1, sg1).wait()
    pltpu.sync_copy(r1, acc_sh.at[dst_v.at[j + 1]], add=True)

    plsc.subcore_barrier()
    pltpu.sync_copy(acc_sh.at[pl.ds(base, NSLICE)],
                    out_hbm.at[cid, pl.ds(base, NSLICE)])


def _edge_body(p_hbm, q_hbm, src_hbm, dst_hbm, w_hbm, blp_hbm, blr_hbm,
               olp_hbm, olr_hbm,
               src_v, dst_v, w_v, blp_v, blr_v,
               u0, v0, u1, v1, t_v, olp_v, olr_v, su0, sv0, su1, sv1):
    cid = lax.axis_index("c")
    tid = lax.axis_index("s")
    wid = cid * NS + tid
    pltpu.sync_copy(src_hbm.at[wid], src_v)
    pltpu.sync_copy(dst_hbm.at[wid], dst_v)
    pltpu.sync_copy(w_hbm, w_v)
    pltpu.sync_copy(blp_hbm, blp_v)
    pltpu.sync_copy(blr_hbm, blr_v)
    ebase = wid * EPT

    def _fire(j, u, v, su, sv):
        pltpu.make_async_copy(p_hbm.at[src_v.at[j]], u, su).start()
        pltpu.make_async_copy(q_hbm.at[dst_v.at[j]], v, sv).start()

    def _wait(j, u, v, su, sv):
        pltpu.make_async_copy(p_hbm.at[src_v.at[j]], u, su).wait()
        pltpu.make_async_copy(q_hbm.at[dst_v.at[j]], v, sv).wait()

    def _compute(j, u, v):
        blp = blp_v[...]
        blr = blr_v[...]
        wregs = [w_v[pl.ds(k * 16, 16)] for k in range(2 * HID // 16)]

        @plsc.parallel_loop(0, CHUNK, 1, unroll=4)
        def _pre(r):
            for k in range(2 * HID // 16):
                sl = pl.ds(k * 16, 16)
                t_v[r, sl] = jnp.maximum(u[r, sl] + v[r, sl], 0.0)

        @plsc.parallel_loop(0, CHUNK // 16, 1, unroll=2)
        def _group(g):
            rows = g * 16 + lax.iota(jnp.int32, 16)
            acc = [jnp.zeros((16,), jnp.float32) for _ in range(4)]
            for f in range(2 * HID):
                cols = jnp.full((16,), f, jnp.int32)
                tf = plsc.load_gather(t_v, [rows, cols])
                term = tf * wregs[f // 16][f % 16]
                slot = (0 if f < HID else 2) + (f & 1)
                acc[slot] = acc[slot] + term
            base16 = g * 16
            alp = acc[0] + acc[1]
            alr = acc[2] + acc[3]
            olp_v[pl.ds(base16, 16)] = 1.0 / (1.0 + jnp.exp(-(alp + blp)))
            olr_v[pl.ds(base16, 16)] = alr + blr
        pltpu.sync_copy(olp_v, olp_hbm.at[pl.ds(ebase + j * CHUNK, CHUNK)])
        pltpu.sync_copy(olr_v, olr_hbm.at[pl.ds(ebase + j * CHUNK, CHUNK)])

    _fire(0, u0, v0, su0, sv0)
    _fire(1, u1, v1, su1, sv1)

    def _pair(k, carry):
        j = 2 * k
        _wait(j, u0, v0, su0, sv0)
        _compute(j, u0, v0)
        _fire(j + 2, u0, v0, su0, sv0)
        _wait(j + 1, u1, v1, su1, sv1)
        _compute(j + 1, u1, v1)
        _fire(j + 3, u1, v1, su1, sv1)
        return carry

    lax.fori_loop(0, NCHUNK // 2 - 1, _pair, 0)
    j = NCHUNK - 2
    _wait(j, u0, v0, su0, sv0)
    _compute(j, u0, v0)
    _wait(j + 1, u1, v1, su1, sv1)
    _compute(j + 1, u1, v1)


_BN = 256
_BE = 1280


def _k1_body(x_ref, w_ref, deg_ref, g_ref):
    dinv = lax.rsqrt(deg_ref[...])
    g_ref[...] = jnp.dot(x_ref[...], w_ref[...],
                         preferred_element_type=jnp.float32) * dinv


def _k2_body(a0_ref, a1_ref, g_ref, deg_ref, w_ref, b_ref, o_ref):
    dinv = lax.rsqrt(deg_ref[...])
    h1 = (a0_ref[...] + a1_ref[...] + g_ref[...]) * dinv + b_ref[...]
    h1 = jnp.maximum(h1, 0.0)
    o_ref[...] = jnp.dot(h1, w_ref[...],
                         preferred_element_type=jnp.float32) * dinv


def _k3_body(a0_ref, a1_ref, g_ref, deg_ref, b2_ref, wp_ref, wq_ref,
             b1c_ref, p_ref, q_ref):
    dinv = lax.rsqrt(deg_ref[...])
    h2 = (a0_ref[...] + a1_ref[...] + g_ref[...]) * dinv + b2_ref[...]
    p_ref[...] = jnp.dot(h2, wp_ref[...], preferred_element_type=jnp.float32)
    q_ref[...] = jnp.dot(h2, wq_ref[...],
                         preferred_element_type=jnp.float32) + b1c_ref[...]


def _k4_body(uv_ref, wlp_ref, wlr_ref, blp_ref, blr_ref,
             lp_ref, lr_ref):
    uv = uv_ref[...]
    t = jnp.maximum(uv[:, :2 * HID] + uv[:, 2 * HID:], 0.0)
    lp = jnp.sum(t[:, :HID] * wlp_ref[...], axis=1, keepdims=True)
    lr = jnp.sum(t[:, HID:] * wlr_ref[...], axis=1, keepdims=True)
    lp_ref[...] = jax.nn.sigmoid(lp + blp_ref[...])
    lr_ref[...] = lr + blr_ref[...]


def kernel(x, edge_index, conv1_W, conv1_b, conv2_W, conv2_b,
           lp_W1, lp_b1, lp_W2, lp_b2, lr_W1, lr_b1, lr_W2, lr_b2):
    f32 = jnp.float32
    x_pad = jnp.pad(x.astype(f32), ((0, NPAD - N), (0, 0)))
    src = edge_index[0].astype(jnp.int32)
    dst = edge_index[1].astype(jnp.int32)
    pad_idx = jnp.full((EPAD - E,), N, jnp.int32)
    src_r = jnp.concatenate([src, pad_idx]).reshape(NTILES, NCHUNK, CHUNK)
    dst_r = jnp.concatenate([dst, pad_idx]).reshape(NTILES, NCHUNK, CHUNK)
    zeros_n = jnp.zeros((NPAD, HID), f32)
    zeros_d = jnp.zeros((NPAD, DEGW), f32)
    ones_d = jnp.ones((CHUNK, DEGW), f32)
    _deg_kernel, _agg_kernel, _edge_kernel = _sc_kernels()

    degp = _deg_kernel(dst_r, ones_d, zeros_d)
    deg_b = jnp.broadcast_to(
        (degp[0, :, 0] + degp[1, :, 0] + 1.0)[:, None], (NPAD, HID))

    grid_n = NPAD // _BN
    bn = lambda i: (i, 0)
    b0 = lambda i: (0, 0)
    spec_n = pl.BlockSpec((_BN, HID), bn)
    spec_deg = pl.BlockSpec((_BN, HID), bn)

    g1 = pl.pallas_call(
        _k1_body,
        grid=(grid_n,),
        in_specs=[pl.BlockSpec((_BN, IN_CH), bn),
                  pl.BlockSpec((IN_CH, HID), b0),
                  spec_deg],
        out_specs=spec_n,
        out_shape=jax.ShapeDtypeStruct((NPAD, HID), f32),
    )(x_pad, conv1_W.astype(f32), deg_b)

    acc1 = _agg_kernel(g1, src_r, dst_r, zeros_n)

    g2 = pl.pallas_call(
        _k2_body,
        grid=(grid_n,),
        in_specs=[spec_n, spec_n, spec_n, spec_deg,
                  pl.BlockSpec((HID, HID), b0),
                  pl.BlockSpec((1, HID), b0)],
        out_specs=spec_n,
        out_shape=jax.ShapeDtypeStruct((NPAD, HID), f32),
    )(acc1[0], acc1[1], g1, deg_b, conv2_W.astype(f32),
      conv1_b.astype(f32).reshape(1, HID))

    acc2 = _agg_kernel(g2, src_r, dst_r, zeros_n)

    WP = jnp.concatenate([lp_W1[:HID], lr_W1[:HID]], axis=1).astype(f32)
    WQ = jnp.concatenate([lp_W1[HID:], lr_W1[HID:]], axis=1).astype(f32)
    b1c = jnp.concatenate([lp_b1, lr_b1]).astype(f32).reshape(1, 2 * HID)
    P, Q = pl.pallas_call(
        _k3_body,
        grid=(grid_n,),
        in_specs=[spec_n, spec_n, spec_n, spec_deg,
                  pl.BlockSpec((1, HID), b0),
                  pl.BlockSpec((HID, 2 * HID), b0),
                  pl.BlockSpec((HID, 2 * HID), b0),
                  pl.BlockSpec((1, 2 * HID), b0)],
        out_specs=[pl.BlockSpec((_BN, 2 * HID), bn),
                   pl.BlockSpec((_BN, 2 * HID), bn)],
        out_shape=[jax.ShapeDtypeStruct((NPAD, 2 * HID), f32),
                   jax.ShapeDtypeStruct((NPAD, 2 * HID), f32)],
    )(acc2[0], acc2[1], g2, deg_b, conv2_b.astype(f32).reshape(1, HID),
      WP, WQ, b1c)

    wcat = jnp.concatenate([lp_W2[:, 0], lr_W2[:, 0]]).astype(f32)
    blp16 = jnp.full((16,), lp_b2[0], f32)
    blr16 = jnp.full((16,), lr_b2[0], f32)
    olp, olr = _edge_kernel(P, Q, src_r, dst_r, wcat, blp16, blr16)

    return (olp[:E][:, None], olr[:E][:, None])


# speedup vs baseline: 1.1562x; 1.1562x over previous
import functools

import jax
import jax.numpy as jnp
from jax import lax
from jax.experimental import pallas as pl
from jax.experimental.pallas import tpu as pltpu
from jax.experimental.pallas import tpu_sc as plsc

N = 10000
E = 320000
IN_CH = 128
HID = 32

NPAD = 10240
EPAD = 327680

NC = 2
NS = 16
NTILES = NC * NS
EPT = EPAD // NTILES
CHUNK = 128
NCHUNK = EPT // CHUNK
NSLICE = NPAD // NS
DEGW = 8

@functools.cache
def _sc_kernels():
    mesh = plsc.VectorSubcoreMesh(core_axis_name="c", subcore_axis_name="s")
    params = pltpu.CompilerParams(use_tc_tiling_on_sc=False)
    params_nl = pltpu.CompilerParams(use_tc_tiling_on_sc=False,
                                     needs_layout_passes=False)
    deg = functools.partial(
        pl.kernel,
        out_type=jax.ShapeDtypeStruct((NC, NPAD, DEGW), jnp.float32),
        mesh=mesh,
        compiler_params=params,
        scratch_types=[
            pltpu.VMEM((NCHUNK, CHUNK), jnp.int32),
            pltpu.VMEM((CHUNK, DEGW), jnp.float32),
            pltpu.VMEM_SHARED((NPAD, DEGW), jnp.float32),
            pltpu.SemaphoreType.DMA,
        ],
    )(_deg_body)
    agg = functools.partial(
        pl.kernel,
        out_type=jax.ShapeDtypeStruct((NC, NPAD, HID), jnp.float32),
        mesh=mesh,
        compiler_params=params,
        scratch_types=[
            pltpu.VMEM((NCHUNK, CHUNK), jnp.int32),
            pltpu.VMEM((NCHUNK, CHUNK), jnp.int32),
            pltpu.VMEM((CHUNK, HID), jnp.float32),
            pltpu.VMEM((CHUNK, HID), jnp.float32),
            pltpu.VMEM_SHARED((NPAD, HID), jnp.float32),
            pltpu.SemaphoreType.DMA,
            pltpu.SemaphoreType.DMA,
            pltpu.SemaphoreType.DMA,
            pltpu.SemaphoreType.DMA,
        ],
    )(_agg_body)
    edge = functools.partial(
        pl.kernel,
        out_type=[
            jax.ShapeDtypeStruct((EPAD,), jnp.float32),
            jax.ShapeDtypeStruct((EPAD,), jnp.float32),
        ],
        mesh=mesh,
        compiler_params=params_nl,
        scratch_types=[
            pltpu.VMEM((NCHUNK, CHUNK), jnp.int32),
            pltpu.VMEM((NCHUNK, CHUNK), jnp.int32),
            pltpu.VMEM((2 * HID,), jnp.float32),
            pltpu.VMEM((16,), jnp.float32),
            pltpu.VMEM((16,), jnp.float32),
            pltpu.VMEM((CHUNK, 2 * HID), jnp.float32),
            pltpu.VMEM((CHUNK, 2 * HID), jnp.float32),
            pltpu.VMEM((CHUNK, 2 * HID), jnp.float32),
            pltpu.VMEM((CHUNK, 2 * HID), jnp.float32),
            pltpu.VMEM((CHUNK, 2 * HID), jnp.float32),
            pltpu.VMEM((CHUNK,), jnp.float32),
            pltpu.VMEM((CHUNK,), jnp.float32),
            pltpu.SemaphoreType.DMA,
            pltpu.SemaphoreType.DMA,
            pltpu.SemaphoreType.DMA,
            pltpu.SemaphoreType.DMA,
        ],
    )(_edge_body)
    return deg, agg, edge


def _deg_body(dst_hbm, ones_hbm, zeros_hbm, out_hbm, dst_v, ones_v, acc_sh,
              sem):
    cid = lax.axis_index("c")
    tid = lax.axis_index("s")
    wid = cid * NS + tid
    base = tid * NSLICE
    pltpu.sync_copy(zeros_hbm.at[pl.ds(base, NSLICE)],
                    acc_sh.at[pl.ds(base, NSLICE)])
    pltpu.sync_copy(ones_hbm, ones_v)
    pltpu.sync_copy(dst_hbm.at[wid], dst_v)
    plsc.subcore_barrier()

    BATCH = 16

    def _batch(b, carry):
        for i in range(BATCH):
            pltpu.make_async_copy(
                ones_v, acc_sh.at[dst_v.at[b * BATCH + i]], sem
            ).start(add=True)
        for i in range(BATCH):
            pltpu.make_async_copy(
                ones_v, acc_sh.at[dst_v.at[b * BATCH + i]], sem
            ).wait()
        return carry

    lax.fori_loop(0, NCHUNK // BATCH, _batch, 0)

    plsc.subcore_barrier()
    pltpu.sync_copy(acc_sh.at[pl.ds(base, NSLICE)],
                    out_hbm.at[cid, pl.ds(base, NSLICE)])


def _agg_body(g_hbm, src_hbm, dst_hbm, zeros_hbm, out_hbm,
              src_v, dst_v, r0, r1, acc_sh, sg0, sg1, ss0, ss1):
    cid = lax.axis_index("c")
    tid = lax.axis_index("s")
    wid = cid * NS + tid
    base = tid * NSLICE
    pltpu.sync_copy(zeros_hbm.at[pl.ds(base, NSLICE)],
                    acc_sh.at[pl.ds(base, NSLICE)])
    pltpu.sync_copy(src_hbm.at[wid], src_v)
    pltpu.sync_copy(dst_hbm.at[wid], dst_v)
    plsc.subcore_barrier()

    def _gather(j, buf, sem):
        return pltpu.make_async_copy(g_hbm.at[src_v.at[j]], buf, sem)

    def _scatter(j, buf, sem):
        return pltpu.make_async_copy(buf, acc_sh.at[dst_v.at[j]], sem)

    _gather(0, r0, sg0).start()
    _gather(1, r1, sg1).start()

    def _pair(k, carry):
        j = 2 * k
        _gather(j, r0, sg0).wait()
        _scatter(j, r0, ss0).start(add=True)
        _gather(j + 1, r1, sg1).wait()
        _scatter(j + 1, r1, ss1).start(add=True)
        _scatter(j, r0, ss0).wait()
        _gather(j + 2, r0, sg0).start()
        _scatter(j + 1, r1, ss1).wait()
        _gather(j + 3, r1, sg1).start()
        return carry

    lax.fori_loop(0, NCHUNK // 2 - 1, _pair, 0)
    j = NCHUNK - 2
    _gather(j, r0, sg0).wait()
    pltpu.sync_copy(r0, acc_sh.at[dst_v.at[j]], add=True)
    _gather(j + 1, r1, sg1).wait()
    pltpu.sync_copy(r1, acc_sh.at[dst_v.at[j + 1]], add=True)

    plsc.subcore_barrier()
    pltpu.sync_copy(acc_sh.at[pl.ds(base, NSLICE)],
                    out_hbm.at[cid, pl.ds(base, NSLICE)])


def _edge_body(p_hbm, q_hbm, src_hbm, dst_hbm, w_hbm, blp_hbm, blr_hbm,
               olp_hbm, olr_hbm,
               src_v, dst_v, w_v, blp_v, blr_v,
               u0, v0, u1, v1, t_v, olp_v, olr_v, su0, sv0, su1, sv1):
    cid = lax.axis_index("c")
    tid = lax.axis_index("s")
    wid = cid * NS + tid
    pltpu.sync_copy(src_hbm.at[wid], src_v)
    pltpu.sync_copy(dst_hbm.at[wid], dst_v)
    pltpu.sync_copy(w_hbm, w_v)
    pltpu.sync_copy(blp_hbm, blp_v)
    pltpu.sync_copy(blr_hbm, blr_v)
    ebase = wid * EPT

    def _fire(j, u, v, su, sv):
        pltpu.make_async_copy(p_hbm.at[src_v.at[j]], u, su).start()
        pltpu.make_async_copy(q_hbm.at[dst_v.at[j]], v, sv).start()

    def _wait(j, u, v, su, sv):
        pltpu.make_async_copy(p_hbm.at[src_v.at[j]], u, su).wait()
        pltpu.make_async_copy(q_hbm.at[dst_v.at[j]], v, sv).wait()

    lane = lax.iota(jnp.int32, 16)
    perms = [lane ^ s for s in (1, 2, 4, 8)]

    def _lanesum(x):
        for p in perms:
            x = x + x.at[p].get(mode="promise_in_bounds")
        return x

    def _compute(j, u, v):
        blp = blp_v[...]
        blr = blr_v[...]
        wregs = [w_v[pl.ds(k * 16, 16)] for k in range(2 * HID // 16)]

        def _edge16(q, carry):
            rlp = jnp.zeros((16,), jnp.float32)
            rlr = jnp.zeros((16,), jnp.float32)
            for l in range(16):
                e = q * 16 + l
                t = [jnp.maximum(u[e, pl.ds(k * 16, 16)]
                                 + v[e, pl.ds(k * 16, 16)], 0.0) * wregs[k]
                     for k in range(2 * HID // 16)]
                slp = _lanesum(t[0] + t[1])
                slr = _lanesum(t[2] + t[3])
                m = lane == l
                rlp = jnp.where(m, slp, rlp)
                rlr = jnp.where(m, slr, rlr)
            base16 = q * 16
            olp_v[pl.ds(base16, 16)] = 1.0 / (1.0 + jnp.exp(-(rlp + blp)))
            olr_v[pl.ds(base16, 16)] = rlr + blr
            return carry

        lax.fori_loop(0, CHUNK // 16, _edge16, 0)
        pltpu.sync_copy(olp_v, olp_hbm.at[pl.ds(ebase + j * CHUNK, CHUNK)])
        pltpu.sync_copy(olr_v, olr_hbm.at[pl.ds(ebase + j * CHUNK, CHUNK)])

    _fire(0, u0, v0, su0, sv0)
    _fire(1, u1, v1, su1, sv1)

    def _pair(k, carry):
        j = 2 * k
        _wait(j, u0, v0, su0, sv0)
        _compute(j, u0, v0)
        _fire(j + 2, u0, v0, su0, sv0)
        _wait(j + 1, u1, v1, su1, sv1)
        _compute(j + 1, u1, v1)
        _fire(j + 3, u1, v1, su1, sv1)
        return carry

    lax.fori_loop(0, NCHUNK // 2 - 1, _pair, 0)
    j = NCHUNK - 2
    _wait(j, u0, v0, su0, sv0)
    _compute(j, u0, v0)
    _wait(j + 1, u1, v1, su1, sv1)
    _compute(j + 1, u1, v1)


_BN = 256
_BE = 1280


def _k1_body(x_ref, w_ref, deg_ref, g_ref):
    dinv = lax.rsqrt(deg_ref[...])
    g_ref[...] = jnp.dot(x_ref[...], w_ref[...],
                         preferred_element_type=jnp.float32) * dinv


def _k2_body(a0_ref, a1_ref, g_ref, deg_ref, w_ref, b_ref, o_ref):
    dinv = lax.rsqrt(deg_ref[...])
    h1 = (a0_ref[...] + a1_ref[...] + g_ref[...]) * dinv + b_ref[...]
    h1 = jnp.maximum(h1, 0.0)
    o_ref[...] = jnp.dot(h1, w_ref[...],
                         preferred_element_type=jnp.float32) * dinv


def _k3_body(a0_ref, a1_ref, g_ref, deg_ref, b2_ref, wp_ref, wq_ref,
             b1c_ref, p_ref, q_ref):
    dinv = lax.rsqrt(deg_ref[...])
    h2 = (a0_ref[...] + a1_ref[...] + g_ref[...]) * dinv + b2_ref[...]
    p_ref[...] = jnp.dot(h2, wp_ref[...], preferred_element_type=jnp.float32)
    q_ref[...] = jnp.dot(h2, wq_ref[...],
                         preferred_element_type=jnp.float32) + b1c_ref[...]


def _k4_body(uv_ref, wlp_ref, wlr_ref, blp_ref, blr_ref,
             lp_ref, lr_ref):
    uv = uv_ref[...]
    t = jnp.maximum(uv[:, :2 * HID] + uv[:, 2 * HID:], 0.0)
    lp = jnp.sum(t[:, :HID] * wlp_ref[...], axis=1, keepdims=True)
    lr = jnp.sum(t[:, HID:] * wlr_ref[...], axis=1, keepdims=True)
    lp_ref[...] = jax.nn.sigmoid(lp + blp_ref[...])
    lr_ref[...] = lr + blr_ref[...]


def kernel(x, edge_index, conv1_W, conv1_b, conv2_W, conv2_b,
           lp_W1, lp_b1, lp_W2, lp_b2, lr_W1, lr_b1, lr_W2, lr_b2):
    f32 = jnp.float32
    x_pad = jnp.pad(x.astype(f32), ((0, NPAD - N), (0, 0)))
    src = edge_index[0].astype(jnp.int32)
    dst = edge_index[1].astype(jnp.int32)
    pad_idx = jnp.full((EPAD - E,), N, jnp.int32)
    src_r = jnp.concatenate([src, pad_idx]).reshape(NTILES, NCHUNK, CHUNK)
    dst_r = jnp.concatenate([dst, pad_idx]).reshape(NTILES, NCHUNK, CHUNK)
    zeros_n = jnp.zeros((NPAD, HID), f32)
    zeros_d = jnp.zeros((NPAD, DEGW), f32)
    ones_d = jnp.ones((CHUNK, DEGW), f32)
    _deg_kernel, _agg_kernel, _edge_kernel = _sc_kernels()

    degp = _deg_kernel(dst_r, ones_d, zeros_d)
    deg_b = jnp.broadcast_to(
        (degp[0, :, 0] + degp[1, :, 0] + 1.0)[:, None], (NPAD, HID))

    grid_n = NPAD // _BN
    bn = lambda i: (i, 0)
    b0 = lambda i: (0, 0)
    spec_n = pl.BlockSpec((_BN, HID), bn)
    spec_deg = pl.BlockSpec((_BN, HID), bn)

    g1 = pl.pallas_call(
        _k1_body,
        grid=(grid_n,),
        in_specs=[pl.BlockSpec((_BN, IN_CH), bn),
                  pl.BlockSpec((IN_CH, HID), b0),
                  spec_deg],
        out_specs=spec_n,
        out_shape=jax.ShapeDtypeStruct((NPAD, HID), f32),
    )(x_pad, conv1_W.astype(f32), deg_b)

    acc1 = _agg_kernel(g1, src_r, dst_r, zeros_n)

    g2 = pl.pallas_call(
        _k2_body,
        grid=(grid_n,),
        in_specs=[spec_n, spec_n, spec_n, spec_deg,
                  pl.BlockSpec((HID, HID), b0),
                  pl.BlockSpec((1, HID), b0)],
        out_specs=spec_n,
        out_shape=jax.ShapeDtypeStruct((NPAD, HID), f32),
    )(acc1[0], acc1[1], g1, deg_b, conv2_W.astype(f32),
      conv1_b.astype(f32).reshape(1, HID))

    acc2 = _agg_kernel(g2, src_r, dst_r, zeros_n)

    WP = jnp.concatenate([lp_W1[:HID], lr_W1[:HID]], axis=1).astype(f32)
    WQ = jnp.concatenate([lp_W1[HID:], lr_W1[HID:]], axis=1).astype(f32)
    b1c = jnp.concatenate([lp_b1, lr_b1]).astype(f32).reshape(1, 2 * HID)
    P, Q = pl.pallas_call(
        _k3_body,
        grid=(grid_n,),
        in_specs=[spec_n, spec_n, spec_n, spec_deg,
                  pl.BlockSpec((1, HID), b0),
                  pl.BlockSpec((HID, 2 * HID), b0),
                  pl.BlockSpec((HID, 2 * HID), b0),
                  pl.BlockSpec((1, 2 * HID), b0)],
        out_specs=[pl.BlockSpec((_BN, 2 * HID), bn),
                   pl.BlockSpec((_BN, 2 * HID), bn)],
        out_shape=[jax.ShapeDtypeStruct((NPAD, 2 * HID), f32),
                   jax.ShapeDtypeStruct((NPAD, 2 * HID), f32)],
    )(acc2[0], acc2[1], g2, deg_b, conv2_b.astype(f32).reshape(1, HID),
      WP, WQ, b1c)

    wcat = jnp.concatenate([lp_W2[:, 0], lr_W2[:, 0]]).astype(f32)
    blp16 = jnp.full((16,), lp_b2[0], f32)
    blr16 = jnp.full((16,), lr_b2[0], f32)
    olp, olr = _edge_kernel(P, Q, src_r, dst_r, wcat, blp16, blr16)

    return (olp[:E][:, None], olr[:E][:, None])
